# baseline (speedup 1.0000x reference)
import functools

import jax
import jax.numpy as jnp
from jax import lax
from jax.experimental import pallas as pl
from jax.experimental.pallas import tpu as pltpu
from jax.experimental.pallas import tpu_sc as plsc

NC = 2
NS = 16
L = 16
NW = NC * NS

CR = 8


def _make_sc_permute(R: int, D: int):
    rows_per_w = R // NW
    n_chunks = rows_per_w // CR
    n_groups = D // L
    chunk_elems = CR * D

    mesh = plsc.VectorSubcoreMesh(
        core_axis_name="c", subcore_axis_name="s",
        num_cores=NC, num_subcores=NS,
    )

    @functools.partial(
        pl.kernel,
        out_type=jax.ShapeDtypeStruct((R * D,), jnp.float32),
        mesh=mesh,
        compiler_params=pltpu.CompilerParams(needs_layout_passes=False),
        scratch_types=[
            pltpu.VMEM((D,), jnp.int32),
            pltpu.VMEM((chunk_elems,), jnp.float32),
            pltpu.VMEM((chunk_elems,), jnp.float32),
            pltpu.VMEM((chunk_elems,), jnp.float32),
            pltpu.VMEM((chunk_elems,), jnp.float32),
            pltpu.SemaphoreType.DMA,
            pltpu.SemaphoreType.DMA,
            pltpu.SemaphoreType.DMA,
            pltpu.SemaphoreType.DMA,
        ],
    )
    def sc_permute(in_hbm, idx_hbm, out_hbm, idx_v,
                   in0, in1, out0, out1, si0, si1, so0, so1):
        wid = lax.axis_index("s") * NC + lax.axis_index("c")
        base = wid * rows_per_w * D
        ins, outs = (in0, in1), (out0, out1)
        sis, sos = (si0, si1), (so0, so1)

        pltpu.sync_copy(idx_hbm, idx_v)

        def start_in(c, b):
            pltpu.async_copy(
                in_hbm.at[pl.ds(base + c * chunk_elems, chunk_elems)],
                ins[b], sis[b])

        def wait_in(b):
            pltpu.make_async_copy(
                in_hbm.at[pl.ds(base, chunk_elems)], ins[b], sis[b]).wait()

        def start_out(c, b):
            pltpu.async_copy(
                outs[b],
                out_hbm.at[pl.ds(base + c * chunk_elems, chunk_elems)],
                sos[b])

        def wait_out(b):
            pltpu.make_async_copy(
                ins[b], out_hbm.at[pl.ds(base, chunk_elems)], sos[b]).wait()

        def compute(in_buf, out_buf):
            def g_body(g, _):
                o = g * L
                col = idx_v[pl.ds(o, L)]
                for r in range(CR):
                    out_buf[pl.ds(r * D + o, L)] = plsc.load_gather(
                        in_buf, [col + jnp.int32(r * D)])
                return 0

            lax.fori_loop(0, n_groups, g_body, 0, unroll=2)

        start_in(0, 0)
        start_in(1, 1)

        def chunk_pair(i, _):
            for b in range(2):
                c = 2 * i + b
                wait_in(b)

                @pl.when(i >= 1)
                def _():
                    wait_out(b)

                compute(ins[b], outs[b])
                start_out(c, b)

                @pl.when(i < n_chunks // 2 - 1)
                def _():
                    start_in(c + 2, b)
            return 0

        lax.fori_loop(0, n_chunks // 2, chunk_pair, 0)
        wait_out(0)
        wait_out(1)

    return sc_permute


def kernel(inputs, idxs):
    B, S, D = inputs.shape
    R = B * S
    x = inputs.reshape(R * D)
    out = _make_sc_permute(R, D)(x, idxs)
    return out.reshape(B, S, D)


# speedup vs baseline: 1.5115x; 1.5115x over previous
import functools

import jax
import jax.numpy as jnp
from jax import lax
from jax.experimental import pallas as pl
from jax.experimental.pallas import tpu as pltpu
from jax.experimental.pallas import tpu_sc as plsc

NC = 2
NS = 16
L = 16
NW = NC * NS

CR = 8


def _make_sc_permute(R: int, D: int):
    rows_per_w = R // NW
    n_chunks = rows_per_w // CR
    n_groups = D // L
    chunk_elems = CR * D

    mesh = plsc.VectorSubcoreMesh(
        core_axis_name="c", subcore_axis_name="s",
        num_cores=NC, num_subcores=NS,
    )

    @functools.partial(
        pl.kernel,
        out_type=jax.ShapeDtypeStruct((R * D,), jnp.float32),
        mesh=mesh,
        compiler_params=pltpu.CompilerParams(needs_layout_passes=False),
        scratch_types=[
            pltpu.VMEM((D,), jnp.int32),
            pltpu.VMEM((chunk_elems,), jnp.float32),
            pltpu.VMEM((chunk_elems,), jnp.float32),
            pltpu.VMEM((chunk_elems,), jnp.float32),
            pltpu.VMEM((chunk_elems,), jnp.float32),
            pltpu.SemaphoreType.DMA,
            pltpu.SemaphoreType.DMA,
            pltpu.SemaphoreType.DMA,
            pltpu.SemaphoreType.DMA,
        ],
    )
    def sc_permute(in_hbm, idx_hbm, out_hbm, idx_v,
                   in0, in1, out0, out1, si0, si1, so0, so1):
        wid = lax.axis_index("s") * NC + lax.axis_index("c")
        base = wid * rows_per_w * D
        ins, outs = (in0, in1), (out0, out1)
        sis, sos = (si0, si1), (so0, so1)

        pltpu.sync_copy(idx_hbm, idx_v)

        def start_in(c, b):
            pltpu.async_copy(
                in_hbm.at[pl.ds(base + c * chunk_elems, chunk_elems)],
                ins[b], sis[b])

        def wait_in(b):
            pltpu.make_async_copy(
                in_hbm.at[pl.ds(base, chunk_elems)], ins[b], sis[b]).wait()

        def start_out(c, b):
            pltpu.async_copy(
                outs[b],
                out_hbm.at[pl.ds(base + c * chunk_elems, chunk_elems)],
                sos[b])

        def wait_out(b):
            pltpu.make_async_copy(
                ins[b], out_hbm.at[pl.ds(base, chunk_elems)], sos[b]).wait()

        def compute(in_buf, out_buf):
            @plsc.parallel_loop(0, n_groups, step=1, unroll=2)
            def _(g):
                o = g * L
                col = idx_v[pl.ds(o, L)]
                vals = [plsc.load_gather(in_buf, [col + jnp.int32(r * D)])
                        for r in range(CR)]
                for r in range(CR):
                    out_buf[pl.ds(r * D + o, L)] = vals[r]

        start_in(0, 0)
        start_in(1, 1)

        def chunk_pair(i, _):
            for b in range(2):
                c = 2 * i + b
                wait_in(b)

                @pl.when(i >= 1)
                def _():
                    wait_out(b)

                compute(ins[b], outs[b])
                start_out(c, b)

                @pl.when(i < n_chunks // 2 - 1)
                def _():
                    start_in(c + 2, b)
            return 0

        lax.fori_loop(0, n_chunks // 2, chunk_pair, 0)
        wait_out(0)
        wait_out(1)

    return sc_permute


def kernel(inputs, idxs):
    B, S, D = inputs.shape
    R = B * S
    x = inputs.reshape(R * D)
    out = _make_sc_permute(R, D)(x, idxs)
    return out.reshape(B, S, D)
